# baseline (speedup 1.0000x reference)
import functools

import jax
import jax.numpy as jnp
from jax import lax
from jax.experimental import pallas as pl
from jax.experimental.pallas import tpu as pltpu
from jax.experimental.pallas import tpu_sc as plsc

NC = 2
NS = 16
NW = NC * NS
LANES = 16
B = 16384
D = 32
BPW = B // NW
NSLOTS = 4
NGRP = BPW // LANES


def _mfmf_body(uid_hbm, iid_hbm, ut_hbm, vt_hbm, it_hbm, zt_hbm, out_hbm,
               uid_v, iid_v, bufs, out_v, sems):
    wid = lax.axis_index("s") * NC + lax.axis_index("c")
    base = wid * BPW

    pltpu.sync_copy(uid_hbm.at[wid], uid_v)
    pltpu.sync_copy(iid_hbm.at[wid], iid_v)

    lane = lax.iota(jnp.int32, LANES)
    perms = [lane ^ (1 << k) for k in range(4)]
    _dnums = lax.GatherDimensionNumbers(
        offset_dims=(), collapsed_slice_dims=(0,), start_index_map=(0,))

    def _permute(x, idx):
        return lax.gather(
            x, idx[:, None], _dnums, (1,),
            mode=lax.GatherScatterMode.PROMISE_IN_BOUNDS)

    def issue(slot, qu, qi):
        ou = pl.multiple_of(qu, 128)
        oi = pl.multiple_of(qi, 128)
        for t, (hbm, off) in enumerate(
                ((ut_hbm, ou), (vt_hbm, oi), (it_hbm, oi), (zt_hbm, ou))):
            for jt in range(D // 8):
                rows = pl.ds(jt * 8, 8)
                pltpu.async_copy(hbm.at[rows, pl.ds(off, 128)],
                                 bufs.at[slot, t, rows], sems.at[slot])

    def wait_slot(slot):
        drain = pltpu.make_async_copy(
            ut_hbm.at[pl.ds(0, 8), pl.ds(0, 128)],
            bufs.at[slot, 0, pl.ds(0, 8)], sems.at[slot])
        for _ in range(16):
            drain.wait()

    u0 = uid_v[pl.ds(0, LANES)]
    i0 = iid_v[pl.ds(0, LANES)]
    qu0 = (u0 >> 7) << 7
    qi0 = (i0 >> 7) << 7
    for l in range(NSLOTS):
        issue(l, qu0[l], qi0[l])

    def group(g, carry):
        s_cur = pl.ds(g * LANES, LANES)
        s_nxt = pl.ds(jnp.minimum(g * LANES + LANES, BPW - LANES), LANES)
        uc = uid_v[s_cur]
        ic = iid_v[s_cur]
        un = uid_v[s_nxt]
        inx = iid_v[s_nxt]
        qu_c, cu_c = (uc >> 7) << 7, uc & 127
        qi_c, ci_c = (ic >> 7) << 7, ic & 127
        qu_n = (un >> 7) << 7
        qi_n = (inx >> 7) << 7

        acc = jnp.zeros((LANES,), jnp.float32)
        for l in range(LANES):
            slot = l % NSLOTS
            wait_slot(slot)

            cu = jnp.full((LANES,), cu_c[l], jnp.int32)
            ci = jnp.full((LANES,), ci_c[l], jnp.int32)
            bu = bufs.at[slot, 0]
            bv = bufs.at[slot, 1]
            bi = bufs.at[slot, 2]
            bz = bufs.at[slot, 3]
            t = (plsc.load_gather(bu, [lane, cu])
                 * plsc.load_gather(bv, [lane, ci])
                 + plsc.load_gather(bi, [lane, ci])
                 * plsc.load_gather(bz, [lane, cu])
                 + plsc.load_gather(bu, [lane + LANES, cu])
                 * plsc.load_gather(bv, [lane + LANES, ci])
                 + plsc.load_gather(bi, [lane + LANES, ci])
                 * plsc.load_gather(bz, [lane + LANES, cu]))
            for p in perms:
                t = t + _permute(t, p)
            acc = jnp.where(lane == l, t, acc)

            m = g * LANES + l + NSLOTS

            @pl.when(m < BPW)
            def _():
                if l + NSLOTS < LANES:
                    issue(slot, qu_c[l + NSLOTS], qi_c[l + NSLOTS])
                else:
                    issue(slot, qu_n[l + NSLOTS - LANES],
                          qi_n[l + NSLOTS - LANES])

        out_v[s_cur] = acc
        return carry

    lax.fori_loop(0, NGRP, group, 0)

    pltpu.sync_copy(out_v, out_hbm.at[pl.ds(base, BPW)])


_mfmf = functools.partial(
    pl.kernel,
    mesh=plsc.VectorSubcoreMesh(core_axis_name="c", subcore_axis_name="s"),
    compiler_params=pltpu.CompilerParams(needs_layout_passes=False),
    out_type=jax.ShapeDtypeStruct((B,), jnp.float32),
    scratch_types=[
        pltpu.VMEM((BPW,), jnp.int32),
        pltpu.VMEM((BPW,), jnp.int32),
        pltpu.VMEM((NSLOTS, 4, D, 128), jnp.float32),
        pltpu.VMEM((BPW,), jnp.float32),
        pltpu.SemaphoreType.DMA((NSLOTS,)),
    ],
)(_mfmf_body)


def kernel(uid, iid, mf_user_emb, mf_item_emb, item_emb, ivae_mean):
    uid2 = uid.reshape(NW, BPW)
    iid2 = iid.reshape(NW, BPW)
    return _mfmf(uid2, iid2, mf_user_emb.T, mf_item_emb.T, item_emb.T,
                 ivae_mean.T)


# speedup vs baseline: 1.0061x; 1.0061x over previous
import functools

import jax
import jax.numpy as jnp
from jax import lax
from jax.experimental import pallas as pl
from jax.experimental.pallas import tpu as pltpu
from jax.experimental.pallas import tpu_sc as plsc

NC = 2
NS = 16
NW = NC * NS
LANES = 16
B = 16384
D = 32
BPW = B // NW
NSLOTS = 4
NGRP = BPW // LANES


def _mfmf_body(uid_hbm, iid_hbm, ut_hbm, vt_hbm, it_hbm, zt_hbm, out_hbm,
               uid_v, iid_v, bufs, out_v, sems):
    wid = lax.axis_index("s") * NC + lax.axis_index("c")
    base = wid * BPW

    pltpu.sync_copy(uid_hbm.at[wid], uid_v)
    pltpu.sync_copy(iid_hbm.at[wid], iid_v)

    lane = lax.iota(jnp.int32, LANES)
    perms = [lane ^ (1 << k) for k in range(4)]
    _dnums = lax.GatherDimensionNumbers(
        offset_dims=(), collapsed_slice_dims=(0,), start_index_map=(0,))

    def _permute(x, idx):
        return lax.gather(
            x, idx[:, None], _dnums, (1,),
            mode=lax.GatherScatterMode.PROMISE_IN_BOUNDS)

    def issue(slot, qu, qi):
        ou = pl.multiple_of(qu, 128)
        oi = pl.multiple_of(qi, 128)
        pltpu.async_copy(ut_hbm.at[:, pl.ds(ou, 128)], bufs.at[slot, 0],
                         sems.at[slot])
        pltpu.async_copy(vt_hbm.at[:, pl.ds(oi, 128)], bufs.at[slot, 1],
                         sems.at[slot])
        pltpu.async_copy(it_hbm.at[:, pl.ds(oi, 128)], bufs.at[slot, 2],
                         sems.at[slot])
        pltpu.async_copy(zt_hbm.at[:, pl.ds(ou, 128)], bufs.at[slot, 3],
                         sems.at[slot])

    def wait_slot(slot):
        drain = pltpu.make_async_copy(
            ut_hbm.at[:, pl.ds(0, 128)], bufs.at[slot, 0], sems.at[slot])
        for _ in range(4):
            drain.wait()

    u0 = uid_v[pl.ds(0, LANES)]
    i0 = iid_v[pl.ds(0, LANES)]
    qu0 = (u0 >> 7) << 7
    qi0 = (i0 >> 7) << 7
    for l in range(NSLOTS):
        issue(l, qu0[l], qi0[l])

    def group(g, carry):
        s_cur = pl.ds(g * LANES, LANES)
        s_nxt = pl.ds(jnp.minimum(g * LANES + LANES, BPW - LANES), LANES)
        uc = uid_v[s_cur]
        ic = iid_v[s_cur]
        un = uid_v[s_nxt]
        inx = iid_v[s_nxt]
        qu_c, cu_c = (uc >> 7) << 7, uc & 127
        qi_c, ci_c = (ic >> 7) << 7, ic & 127
        qu_n = (un >> 7) << 7
        qi_n = (inx >> 7) << 7

        acc = jnp.zeros((LANES,), jnp.float32)
        for l in range(LANES):
            slot = l % NSLOTS
            wait_slot(slot)

            cu = jnp.full((LANES,), cu_c[l], jnp.int32)
            ci = jnp.full((LANES,), ci_c[l], jnp.int32)
            bu = bufs.at[slot, 0]
            bv = bufs.at[slot, 1]
            bi = bufs.at[slot, 2]
            bz = bufs.at[slot, 3]
            t = (plsc.load_gather(bu, [lane, cu])
                 * plsc.load_gather(bv, [lane, ci])
                 + plsc.load_gather(bi, [lane, ci])
                 * plsc.load_gather(bz, [lane, cu])
                 + plsc.load_gather(bu, [lane + LANES, cu])
                 * plsc.load_gather(bv, [lane + LANES, ci])
                 + plsc.load_gather(bi, [lane + LANES, ci])
                 * plsc.load_gather(bz, [lane + LANES, cu]))
            for p in perms:
                t = t + _permute(t, p)
            acc = jnp.where(lane == l, t, acc)

            m = g * LANES + l + NSLOTS

            @pl.when(m < BPW)
            def _():
                if l + NSLOTS < LANES:
                    issue(slot, qu_c[l + NSLOTS], qi_c[l + NSLOTS])
                else:
                    issue(slot, qu_n[l + NSLOTS - LANES],
                          qi_n[l + NSLOTS - LANES])

        out_v[s_cur] = acc
        return carry

    lax.fori_loop(0, NGRP, group, 0)

    pltpu.sync_copy(out_v, out_hbm.at[pl.ds(base, BPW)])


_mfmf = functools.partial(
    pl.kernel,
    mesh=plsc.VectorSubcoreMesh(core_axis_name="c", subcore_axis_name="s"),
    compiler_params=pltpu.CompilerParams(needs_layout_passes=False),
    out_type=jax.ShapeDtypeStruct((B,), jnp.float32),
    scratch_types=[
        pltpu.VMEM((BPW,), jnp.int32),
        pltpu.VMEM((BPW,), jnp.int32),
        pltpu.VMEM((NSLOTS, 4, D, 128), jnp.float32),
        pltpu.VMEM((BPW,), jnp.float32),
        pltpu.SemaphoreType.DMA((NSLOTS,)),
    ],
)(_mfmf_body)


def kernel(uid, iid, mf_user_emb, mf_item_emb, item_emb, ivae_mean):
    uid2 = uid.reshape(NW, BPW)
    iid2 = iid.reshape(NW, BPW)
    return _mfmf(uid2, iid2, mf_user_emb.T, mf_item_emb.T, item_emb.T,
                 ivae_mean.T)
